# baseline (speedup 1.0000x reference)
import functools

import jax
import jax.numpy as jnp
from jax import lax
from jax.experimental import pallas as pl
from jax.experimental.pallas import tpu as pltpu
from jax.experimental.pallas import tpu_sc as plsc

N = 10000
F1 = 128
F2 = 64
NC = 2
NS = 16
CHUNK = 128
SUPER = 40
ROW_BLK = 2000
SUB_ROWS = 624
TAIL_ROWS = N - NS * SUB_ROWS


def _bcast_lane(v16, e):
    return lax.gather(
        v16,
        jnp.full((16, 1), e, jnp.int32),
        lax.GatherDimensionNumbers(
            offset_dims=(), collapsed_slice_dims=(0,), start_index_map=(0,)),
        (1,),
        mode=lax.GatherScatterMode.PROMISE_IN_BOUNDS,
    )


def _make_spmm(e_pad, feat):
    fh = feat // NC
    nvec = fh // 16
    per_t = e_pad // NS
    n_chunks = per_t // CHUNK
    n_super = n_chunks // SUPER
    mesh = plsc.VectorSubcoreMesh(core_axis_name="c", subcore_axis_name="s")

    @functools.partial(
        pl.kernel,
        out_type=jax.ShapeDtypeStruct((NC, N, fh), jnp.float32),
        mesh=mesh,
        scratch_types=[
            pltpu.VMEM((2, SUPER, CHUNK), jnp.int32),
            pltpu.VMEM((SUPER, CHUNK), jnp.float32),
            pltpu.VMEM((2, CHUNK, fh), jnp.float32),
            pltpu.VMEM_SHARED((N, fh), jnp.float32),
            pltpu.VMEM_SHARED((N, fh), jnp.float32),
            [pltpu.SemaphoreType.DMA] * 2,
            [pltpu.SemaphoreType.DMA] * 2,
        ],
        compiler_params=pltpu.CompilerParams(use_tc_tiling_on_sc=False),
    )
    def spmm(table_hbm, src_hbm, dst_hbm, ew_hbm, zeros_hbm, out_hbm,
             idx_v, ew_v, rows_v, table_sp, acc, gsems, ssems):
        c = lax.axis_index("c")
        s = lax.axis_index("s")
        rbase = s * SUB_ROWS
        col0 = c * fh
        pltpu.sync_copy(table_hbm.at[pl.ds(rbase, SUB_ROWS),
                                     pl.ds(col0, fh)],
                        table_sp.at[pl.ds(rbase, SUB_ROWS), :])
        pltpu.sync_copy(zeros_hbm.at[pl.ds(rbase, SUB_ROWS), :],
                        acc.at[pl.ds(rbase, SUB_ROWS), :])

        @pl.when(s == NS - 1)
        def _():
            pltpu.sync_copy(table_hbm.at[pl.ds(NS * SUB_ROWS, TAIL_ROWS),
                                         pl.ds(col0, fh)],
                            table_sp.at[pl.ds(NS * SUB_ROWS, TAIL_ROWS), :])
            pltpu.sync_copy(zeros_hbm.at[pl.ds(NS * SUB_ROWS, TAIL_ROWS), :],
                            acc.at[pl.ds(NS * SUB_ROWS, TAIL_ROWS), :])

        plsc.subcore_barrier()

        def start_gather(k, b):
            pltpu.async_copy(table_sp.at[idx_v.at[0, k]], rows_v.at[b],
                             gsems[b])

        def wait_gather(k, b):
            pltpu.make_async_copy(table_sp.at[idx_v.at[0, k]], rows_v.at[b],
                                  gsems[b]).wait()

        def start_scatter(k, b):
            pltpu.async_copy(rows_v.at[b], acc.at[idx_v.at[1, k]], ssems[b],
                             add=True)

        def wait_scatter(k, b):
            pltpu.make_async_copy(rows_v.at[b], acc.at[idx_v.at[1, k]],
                                  ssems[b]).wait()

        def scale(k, b):
            @plsc.parallel_loop(0, CHUNK // 16, 1, unroll=4)
            def _group(g):
                w16 = ew_v[k, pl.ds(g * 16, 16)]
                for e in range(16):
                    w = _bcast_lane(w16, e)
                    row = g * 16 + e
                    for f in range(nvec):
                        rows_v[b, row, pl.ds(f * 16, 16)] = (
                            rows_v[b, row, pl.ds(f * 16, 16)] * w)

        for sp in range(n_super):
            if sp > 0:
                wait_scatter(SUPER - 1, 1)
            pltpu.sync_copy(src_hbm.at[s, sp], idx_v.at[0])
            pltpu.sync_copy(dst_hbm.at[s, sp], idx_v.at[1])
            pltpu.sync_copy(ew_hbm.at[s, sp], ew_v)
            start_gather(0, 0)

            def pair_body(p, carry):
                k0 = 2 * p

                @pl.when(p > 0)
                def _():
                    wait_scatter(k0 - 1, 1)

                start_gather(k0 + 1, 1)
                wait_gather(k0, 0)
                scale(k0, 0)
                start_scatter(k0, 0)
                wait_gather(k0 + 1, 1)
                scale(k0 + 1, 1)
                start_scatter(k0 + 1, 1)
                wait_scatter(k0, 0)

                @pl.when(p < SUPER // 2 - 1)
                def _():
                    start_gather(k0 + 2, 0)

                return carry

            lax.fori_loop(0, SUPER // 2, pair_body, 0)

        wait_scatter(SUPER - 1, 1)
        plsc.subcore_barrier()
        pltpu.sync_copy(acc.at[pl.ds(rbase, SUB_ROWS), :],
                        out_hbm.at[c, pl.ds(rbase, SUB_ROWS), :])

        @pl.when(s == NS - 1)
        def _():
            pltpu.sync_copy(acc.at[pl.ds(NS * SUB_ROWS, TAIL_ROWS), :],
                            out_hbm.at[c, pl.ds(NS * SUB_ROWS, TAIL_ROWS), :])

    return spmm


_spmm_f1 = None
_spmm_f2 = None


def _get_spmm(e_pad, feat):
    global _spmm_f1, _spmm_f2
    if feat == F1:
        if _spmm_f1 is None:
            _spmm_f1 = _make_spmm(e_pad, feat)
        return _spmm_f1
    if _spmm_f2 is None:
        _spmm_f2 = _make_spmm(e_pad, feat)
    return _spmm_f2


def _mm1(x, W1):
    def body(x_ref, w_ref, o_ref):
        o_ref[...] = jnp.dot(x_ref[...], w_ref[...],
                             preferred_element_type=jnp.float32)

    return pl.pallas_call(
        body,
        grid=(N // ROW_BLK,),
        in_specs=[
            pl.BlockSpec((ROW_BLK, F1), lambda i: (i, 0)),
            pl.BlockSpec((F1, F1), lambda i: (0, 0)),
        ],
        out_specs=pl.BlockSpec((ROW_BLK, F1), lambda i: (i, 0)),
        out_shape=jax.ShapeDtypeStruct((N, F1), jnp.float32),
    )(x, W1)


def _relu_mm2(p, b1, W2):
    def body(p_ref, b_ref, w_ref, o_ref):
        h = jnp.concatenate([p_ref[0], p_ref[1]], axis=1)
        h = jax.nn.relu(h + b_ref[...])
        o_ref[...] = jnp.dot(h, w_ref[...], preferred_element_type=jnp.float32)

    fh = F1 // NC
    return pl.pallas_call(
        body,
        grid=(N // ROW_BLK,),
        in_specs=[
            pl.BlockSpec((NC, ROW_BLK, fh), lambda i: (0, i, 0)),
            pl.BlockSpec((1, F1), lambda i: (0, 0)),
            pl.BlockSpec((F1, F2), lambda i: (0, 0)),
        ],
        out_specs=pl.BlockSpec((ROW_BLK, F2), lambda i: (i, 0)),
        out_shape=jax.ShapeDtypeStruct((N, F2), jnp.float32),
    )(p, b1.reshape(1, F1), W2)


def _logsoftmax_head(q, b2):
    def body(q_ref, b_ref, o_ref):
        z = jnp.concatenate([q_ref[0], q_ref[1]], axis=1) + b_ref[...]
        m = jnp.max(z, axis=1, keepdims=True)
        ez = jnp.exp(z - m)
        ssum = jnp.sum(ez, axis=1, keepdims=True)
        o_ref[...] = z - m - jnp.log(ssum)

    fh = F2 // NC
    return pl.pallas_call(
        body,
        grid=(N // ROW_BLK,),
        in_specs=[
            pl.BlockSpec((NC, ROW_BLK, fh), lambda i: (0, i, 0)),
            pl.BlockSpec((1, F2), lambda i: (0, 0)),
        ],
        out_specs=pl.BlockSpec((ROW_BLK, F2), lambda i: (i, 0)),
        out_shape=jax.ShapeDtypeStruct((N, F2), jnp.float32),
    )(q, b2.reshape(1, F2))


@jax.jit
def kernel(x, edge_index, edge_weight, W1, b1, W2, b2):
    e = edge_weight.shape[0]
    blk = NS * SUPER * CHUNK
    e_pad = ((e + blk - 1) // blk) * blk
    pad = e_pad - e
    n_super = e_pad // blk
    shape4 = (NS, n_super, SUPER, CHUNK)
    src = jnp.pad(edge_index[0].astype(jnp.int32), (0, pad)).reshape(shape4)
    dst = jnp.pad(edge_index[1].astype(jnp.int32), (0, pad)).reshape(shape4)
    ew = jnp.pad(edge_weight, (0, pad)).reshape(shape4)

    zeros1 = jnp.zeros((N, F1 // NC), jnp.float32)
    zeros2 = jnp.zeros((N, F2 // NC), jnp.float32)

    support = _mm1(x, W1)
    p = _get_spmm(e_pad, F1)(support, src, dst, ew, zeros1)
    support2 = _relu_mm2(p, b1, W2)
    q = _get_spmm(e_pad, F2)(support2, src, dst, ew, zeros2)
    return _logsoftmax_head(q, b2)


# speedup vs baseline: 1.0076x; 1.0076x over previous
import functools

import jax
import jax.numpy as jnp
from jax import lax
from jax.experimental import pallas as pl
from jax.experimental.pallas import tpu as pltpu
from jax.experimental.pallas import tpu_sc as plsc

N = 10000
F1 = 128
F2 = 64
NC = 2
NS = 16
CHUNK = 128
SUPER = 40
ROW_BLK = 2000
SUB_ROWS = 624
TAIL_ROWS = N - NS * SUB_ROWS


def _bcast_lane(v16, e):
    return lax.gather(
        v16,
        jnp.full((16, 1), e, jnp.int32),
        lax.GatherDimensionNumbers(
            offset_dims=(), collapsed_slice_dims=(0,), start_index_map=(0,)),
        (1,),
        mode=lax.GatherScatterMode.PROMISE_IN_BOUNDS,
    )


def _make_spmm(e_pad, feat):
    fh = feat // NC
    nvec = fh // 16
    per_t = e_pad // NS
    n_chunks = per_t // CHUNK
    n_super = n_chunks // SUPER
    mesh = plsc.VectorSubcoreMesh(core_axis_name="c", subcore_axis_name="s")

    @functools.partial(
        pl.kernel,
        out_type=jax.ShapeDtypeStruct((NC, N, fh), jnp.float32),
        mesh=mesh,
        scratch_types=[
            pltpu.VMEM((2, SUPER, CHUNK), jnp.int32),
            pltpu.VMEM((SUPER, CHUNK), jnp.float32),
            pltpu.VMEM((2, CHUNK, fh), jnp.float32),
            pltpu.VMEM_SHARED((N, fh), jnp.float32),
            pltpu.VMEM_SHARED((N, fh), jnp.float32),
            [pltpu.SemaphoreType.DMA] * 2,
            [pltpu.SemaphoreType.DMA] * 2,
        ],
        compiler_params=pltpu.CompilerParams(use_tc_tiling_on_sc=False),
    )
    def spmm(table_hbm, src_hbm, dst_hbm, ew_hbm, out_hbm,
             idx_v, ew_v, rows_v, table_sp, acc, gsems, ssems):
        c = lax.axis_index("c")
        s = lax.axis_index("s")
        rbase = s * SUB_ROWS
        col0 = c * fh
        pltpu.sync_copy(table_hbm.at[pl.ds(rbase, SUB_ROWS),
                                     pl.ds(col0, fh)],
                        table_sp.at[pl.ds(rbase, SUB_ROWS), :])

        @plsc.parallel_loop(0, CHUNK, 1, unroll=4)
        def _zero(r):
            for f in range(nvec):
                rows_v[0, r, pl.ds(f * 16, 16)] = jnp.zeros((16,), jnp.float32)

        for i in range(SUB_ROWS // CHUNK):
            pltpu.sync_copy(rows_v.at[0],
                            acc.at[pl.ds(rbase + i * CHUNK, CHUNK), :])
        rem = SUB_ROWS % CHUNK
        pltpu.sync_copy(rows_v.at[0, pl.ds(0, rem)],
                        acc.at[pl.ds(rbase + SUB_ROWS - rem, rem), :])

        @pl.when(s == NS - 1)
        def _():
            pltpu.sync_copy(table_hbm.at[pl.ds(NS * SUB_ROWS, TAIL_ROWS),
                                         pl.ds(col0, fh)],
                            table_sp.at[pl.ds(NS * SUB_ROWS, TAIL_ROWS), :])
            pltpu.sync_copy(rows_v.at[0, pl.ds(0, TAIL_ROWS)],
                            acc.at[pl.ds(NS * SUB_ROWS, TAIL_ROWS), :])

        plsc.subcore_barrier()

        def start_gather(k, b):
            pltpu.async_copy(table_sp.at[idx_v.at[0, k]], rows_v.at[b],
                             gsems[b])

        def wait_gather(k, b):
            pltpu.make_async_copy(table_sp.at[idx_v.at[0, k]], rows_v.at[b],
                                  gsems[b]).wait()

        def start_scatter(k, b):
            pltpu.async_copy(rows_v.at[b], acc.at[idx_v.at[1, k]], ssems[b],
                             add=True)

        def wait_scatter(k, b):
            pltpu.make_async_copy(rows_v.at[b], acc.at[idx_v.at[1, k]],
                                  ssems[b]).wait()

        def scale(k, b):
            @plsc.parallel_loop(0, CHUNK // 16, 1, unroll=4)
            def _group(g):
                w16 = ew_v[k, pl.ds(g * 16, 16)]
                for e in range(16):
                    w = _bcast_lane(w16, e)
                    row = g * 16 + e
                    for f in range(nvec):
                        rows_v[b, row, pl.ds(f * 16, 16)] = (
                            rows_v[b, row, pl.ds(f * 16, 16)] * w)

        for sp in range(n_super):
            if sp > 0:
                wait_scatter(SUPER - 1, 1)
            pltpu.sync_copy(src_hbm.at[s, sp], idx_v.at[0])
            pltpu.sync_copy(dst_hbm.at[s, sp], idx_v.at[1])
            pltpu.sync_copy(ew_hbm.at[s, sp], ew_v)
            start_gather(0, 0)

            def pair_body(p, carry):
                k0 = 2 * p

                @pl.when(p > 0)
                def _():
                    wait_scatter(k0 - 1, 1)

                start_gather(k0 + 1, 1)
                wait_gather(k0, 0)
                scale(k0, 0)
                start_scatter(k0, 0)
                wait_gather(k0 + 1, 1)
                scale(k0 + 1, 1)
                start_scatter(k0 + 1, 1)
                wait_scatter(k0, 0)

                @pl.when(p < SUPER // 2 - 1)
                def _():
                    start_gather(k0 + 2, 0)

                return carry

            lax.fori_loop(0, SUPER // 2, pair_body, 0)

        wait_scatter(SUPER - 1, 1)
        plsc.subcore_barrier()
        pltpu.sync_copy(acc.at[pl.ds(rbase, SUB_ROWS), :],
                        out_hbm.at[c, pl.ds(rbase, SUB_ROWS), :])

        @pl.when(s == NS - 1)
        def _():
            pltpu.sync_copy(acc.at[pl.ds(NS * SUB_ROWS, TAIL_ROWS), :],
                            out_hbm.at[c, pl.ds(NS * SUB_ROWS, TAIL_ROWS), :])

    return spmm


_spmm_f1 = None
_spmm_f2 = None


def _get_spmm(e_pad, feat):
    global _spmm_f1, _spmm_f2
    if feat == F1:
        if _spmm_f1 is None:
            _spmm_f1 = _make_spmm(e_pad, feat)
        return _spmm_f1
    if _spmm_f2 is None:
        _spmm_f2 = _make_spmm(e_pad, feat)
    return _spmm_f2


def _mm1(x, W1):
    def body(x_ref, w_ref, o_ref):
        o_ref[...] = jnp.dot(x_ref[...], w_ref[...],
                             preferred_element_type=jnp.float32)

    return pl.pallas_call(
        body,
        grid=(N // ROW_BLK,),
        in_specs=[
            pl.BlockSpec((ROW_BLK, F1), lambda i: (i, 0)),
            pl.BlockSpec((F1, F1), lambda i: (0, 0)),
        ],
        out_specs=pl.BlockSpec((ROW_BLK, F1), lambda i: (i, 0)),
        out_shape=jax.ShapeDtypeStruct((N, F1), jnp.float32),
    )(x, W1)


def _relu_mm2(p, b1, W2):
    def body(p_ref, b_ref, w_ref, o_ref):
        h = jnp.concatenate([p_ref[0], p_ref[1]], axis=1)
        h = jax.nn.relu(h + b_ref[...])
        o_ref[...] = jnp.dot(h, w_ref[...], preferred_element_type=jnp.float32)

    fh = F1 // NC
    return pl.pallas_call(
        body,
        grid=(N // ROW_BLK,),
        in_specs=[
            pl.BlockSpec((NC, ROW_BLK, fh), lambda i: (0, i, 0)),
            pl.BlockSpec((1, F1), lambda i: (0, 0)),
            pl.BlockSpec((F1, F2), lambda i: (0, 0)),
        ],
        out_specs=pl.BlockSpec((ROW_BLK, F2), lambda i: (i, 0)),
        out_shape=jax.ShapeDtypeStruct((N, F2), jnp.float32),
    )(p, b1.reshape(1, F1), W2)


def _logsoftmax_head(q, b2):
    def body(q_ref, b_ref, o_ref):
        z = jnp.concatenate([q_ref[0], q_ref[1]], axis=1) + b_ref[...]
        m = jnp.max(z, axis=1, keepdims=True)
        ez = jnp.exp(z - m)
        ssum = jnp.sum(ez, axis=1, keepdims=True)
        o_ref[...] = z - m - jnp.log(ssum)

    fh = F2 // NC
    return pl.pallas_call(
        body,
        grid=(N // ROW_BLK,),
        in_specs=[
            pl.BlockSpec((NC, ROW_BLK, fh), lambda i: (0, i, 0)),
            pl.BlockSpec((1, F2), lambda i: (0, 0)),
        ],
        out_specs=pl.BlockSpec((ROW_BLK, F2), lambda i: (i, 0)),
        out_shape=jax.ShapeDtypeStruct((N, F2), jnp.float32),
    )(q, b2.reshape(1, F2))


@jax.jit
def kernel(x, edge_index, edge_weight, W1, b1, W2, b2):
    e = edge_weight.shape[0]
    blk = NS * SUPER * CHUNK
    e_pad = ((e + blk - 1) // blk) * blk
    pad = e_pad - e
    n_super = e_pad // blk
    shape4 = (NS, n_super, SUPER, CHUNK)
    src = jnp.pad(edge_index[0].astype(jnp.int32), (0, pad)).reshape(shape4)
    dst = jnp.pad(edge_index[1].astype(jnp.int32), (0, pad)).reshape(shape4)
    ew = jnp.pad(edge_weight, (0, pad)).reshape(shape4)

    support = _mm1(x, W1)
    p = _get_spmm(e_pad, F1)(support, src, dst, ew)
    support2 = _relu_mm2(p, b1, W2)
    q = _get_spmm(e_pad, F2)(support2, src, dst, ew)
    return _logsoftmax_head(q, b2)
